# baseline (speedup 1.0000x reference)
import jax
import jax.numpy as jnp
from jax.experimental import pallas as pl

N_NODES = 10000
MAX_DEG = 32
D_FEAT = 128
_BN = 200
_INV = 1.0 / MAX_DEG


def _mean_body(x_ref, o_ref):
    o_ref[...] = jnp.sum(x_ref[...], axis=1) * _INV


def kernel(mailbox_m):
    return pl.pallas_call(
        _mean_body,
        grid=(N_NODES // _BN,),
        in_specs=[pl.BlockSpec((_BN, MAX_DEG, D_FEAT), lambda i: (i, 0, 0))],
        out_specs=pl.BlockSpec((_BN, D_FEAT), lambda i: (i, 0)),
        out_shape=jax.ShapeDtypeStruct((N_NODES, D_FEAT), jnp.float32),
    )(mailbox_m)
